# baseline (speedup 1.0000x reference)
import functools

import jax
import jax.numpy as jnp
from jax import lax
from jax.experimental import pallas as pl
from jax.experimental.pallas import tpu as pltpu
from jax.experimental.pallas import tpu_sc as plsc

_B = 16384
_F = 26
_C = 100
_W = _F * _C
_NC = 2
_NS = 16
_L = 16
_ROWS_PER_C = _B // _NC
_RB = 256
_NIT = _ROWS_PER_C // _RB
_RT = _RB // _NS
_XCH = _RT * _F
_OCH = _RB * _W
_NGRP = _XCH // _L


def _body(x_hbm, common_hbm, out_hbm, img0, img1, zb, xb0, xb1, ib0, ib1,
          cb, ones_v, zeros_v, sem0, sem1):
    c = lax.axis_index("c")
    s = lax.axis_index("s")
    crow = c * _ROWS_PER_C
    sbase = s * (_RT * _W)

    pltpu.sync_copy(common_hbm, cb)

    zeros = jnp.zeros((_L,), jnp.int32)
    ones = jnp.full((_L,), 1, jnp.int32)
    for g in range(_NGRP):
        ones_v[pl.ds(g * _L, _L)] = ones
        zeros_v[pl.ds(g * _L, _L)] = zeros

    def zfill(i, carry):
        for k in range(8):
            zb[pl.ds(i * (8 * _L) + k * _L, _L)] = zeros
        return carry

    lax.fori_loop(0, (_RT * _W) // (8 * _L), zfill, 0)
    pltpu.sync_copy(zb, img0.at[pl.ds(sbase, _RT * _W)])
    pltpu.sync_copy(zb, img1.at[pl.ds(sbase, _RT * _W)])
    plsc.subcore_barrier()

    imgs = (img0, img1)
    xbs = (xb0, xb1)
    ibs = (ib0, ib1)
    sems = (sem0, sem1)

    def build_idx(slot, it):
        xoff = (crow + it * _RB) * _F + s * _XCH
        pltpu.sync_copy(x_hbm.at[pl.ds(xoff, _XCH)], xbs[slot])
        for g in range(_NGRP):
            xv = xbs[slot][pl.ds(g * _L, _L)]
            cv = cb[pl.ds(g * _L, _L)]
            ibs[slot][pl.ds(g * _L, _L)] = xv + cv + sbase

    def scatter(slot, val_v):
        pltpu.sync_copy(val_v, imgs[slot].at[ibs[slot]])

    def start_out(slot, it):
        pltpu.make_async_copy(
            imgs[slot],
            out_hbm.at[pl.ds((crow + it * _RB) * _W, _OCH)],
            sems[slot],
        ).start()

    def wait_out(slot):
        pltpu.make_async_copy(
            imgs[slot], out_hbm.at[pl.ds(crow * _W, _OCH)], sems[slot]
        ).wait()

    for slot in range(2):
        build_idx(slot, slot)
        scatter(slot, ones_v)
        plsc.subcore_barrier()

        @pl.when(s == 0)
        def _():
            start_out(slot, slot)

    def body(i, carry):
        it0 = 2 + i * 2
        for slot in range(2):
            it = it0 + slot

            @pl.when(s == 0)
            def _():
                wait_out(slot)

            plsc.subcore_barrier()
            scatter(slot, zeros_v)
            build_idx(slot, it)
            scatter(slot, ones_v)
            plsc.subcore_barrier()

            @pl.when(s == 0)
            def _():
                start_out(slot, it)

        return carry

    lax.fori_loop(0, (_NIT - 2) // 2, body, 0)

    @pl.when(s == 0)
    def _():
        for slot in range(2):
            wait_out(slot)

    plsc.subcore_barrier()


@functools.partial(
    pl.kernel,
    out_type=jax.ShapeDtypeStruct((_B * _W,), jnp.int32),
    mesh=plsc.VectorSubcoreMesh(core_axis_name="c", subcore_axis_name="s"),
    compiler_params=pltpu.CompilerParams(needs_layout_passes=False),
    scratch_types=[
        pltpu.VMEM_SHARED((_OCH,), jnp.int32),
        pltpu.VMEM_SHARED((_OCH,), jnp.int32),
        pltpu.VMEM((_RT * _W,), jnp.int32),
        pltpu.VMEM((_XCH,), jnp.int32),
        pltpu.VMEM((_XCH,), jnp.int32),
        pltpu.VMEM((_XCH,), jnp.int32),
        pltpu.VMEM((_XCH,), jnp.int32),
        pltpu.VMEM((_XCH,), jnp.int32),
        pltpu.VMEM((_XCH,), jnp.int32),
        pltpu.VMEM((_XCH,), jnp.int32),
        pltpu.SemaphoreType.DMA,
        pltpu.SemaphoreType.DMA,
    ],
)
def _onehot_sc(x_hbm, common_hbm, out_hbm, img0, img1, zb, xb0, xb1, ib0,
               ib1, cb, ones_v, zeros_v, sem0, sem1):
    _body(x_hbm, common_hbm, out_hbm, img0, img1, zb, xb0, xb1, ib0, ib1,
          cb, ones_v, zeros_v, sem0, sem1)


def kernel(x):
    t = jnp.arange(_XCH, dtype=jnp.int32)
    common = (t // _F) * _W + (t % _F) * _C
    return _onehot_sc(x.reshape(-1), common).reshape(_B, _W)


# speedup vs baseline: 1.0197x; 1.0197x over previous
import functools

import jax
import jax.numpy as jnp
from jax import lax
from jax.experimental import pallas as pl
from jax.experimental.pallas import tpu as pltpu
from jax.experimental.pallas import tpu_sc as plsc

_B = 16384
_F = 26
_C = 100
_W = _F * _C
_NC = 2
_NS = 16
_L = 16
_ROWS_PER_C = _B // _NC
_RB = 256
_NIT = _ROWS_PER_C // _RB
_RT = _RB // _NS
_XCH = _RT * _F
_OCH = _RB * _W
_NGRP = _XCH // _L


def _body(x_hbm, common_hbm, out_hbm, img0, img1, zb, xb0, xb1, ib0, ib1,
          cb, ones_v, zeros_v, sem0, sem1):
    c = lax.axis_index("c")
    s = lax.axis_index("s")
    crow = c * _ROWS_PER_C
    sbase = s * (_RT * _W)

    pltpu.sync_copy(common_hbm, cb)

    zeros = jnp.zeros((_L,), jnp.int32)
    ones = jnp.full((_L,), 1, jnp.int32)
    for g in range(_NGRP):
        ones_v[pl.ds(g * _L, _L)] = ones
        zeros_v[pl.ds(g * _L, _L)] = zeros

    def zfill(i, carry):
        for k in range(8):
            zb[pl.ds(i * (8 * _L) + k * _L, _L)] = zeros
        return carry

    lax.fori_loop(0, (_RT * _W) // (8 * _L), zfill, 0)
    pltpu.sync_copy(zb, img0.at[pl.ds(sbase, _RT * _W)])
    pltpu.sync_copy(zb, img1.at[pl.ds(sbase, _RT * _W)])
    plsc.subcore_barrier()

    imgs = (img0, img1)
    xbs = (xb0, xb1)
    ibs = (ib0, ib1)
    sems = (sem0, sem1)

    def build_idx(slot, it):
        xoff = (crow + it * _RB) * _F + s * _XCH
        pltpu.sync_copy(x_hbm.at[pl.ds(xoff, _XCH)], xbs[slot])
        for g in range(_NGRP):
            xv = xbs[slot][pl.ds(g * _L, _L)]
            cv = cb[pl.ds(g * _L, _L)]
            ibs[slot][pl.ds(g * _L, _L)] = xv + cv + sbase

    def scatter(slot, val_v):
        pltpu.sync_copy(val_v, imgs[slot].at[ibs[slot]])

    def start_out(slot, it):
        pltpu.make_async_copy(
            imgs[slot].at[pl.ds(sbase, _RT * _W)],
            out_hbm.at[pl.ds((crow + it * _RB) * _W + sbase, _RT * _W)],
            sems[slot],
        ).start()

    def wait_out(slot):
        pltpu.make_async_copy(
            imgs[slot].at[pl.ds(sbase, _RT * _W)],
            out_hbm.at[pl.ds(crow * _W + sbase, _RT * _W)],
            sems[slot],
        ).wait()

    for slot in range(2):
        build_idx(slot, slot)
        scatter(slot, ones_v)
        plsc.subcore_barrier()
        start_out(slot, slot)

    def body(i, carry):
        it0 = 2 + i * 2
        for slot in range(2):
            it = it0 + slot
            wait_out(slot)
            scatter(slot, zeros_v)
            build_idx(slot, it)
            scatter(slot, ones_v)
            start_out(slot, it)
        return carry

    lax.fori_loop(0, (_NIT - 2) // 2, body, 0)

    for slot in range(2):
        wait_out(slot)


@functools.partial(
    pl.kernel,
    out_type=jax.ShapeDtypeStruct((_B * _W,), jnp.int32),
    mesh=plsc.VectorSubcoreMesh(core_axis_name="c", subcore_axis_name="s"),
    compiler_params=pltpu.CompilerParams(needs_layout_passes=False),
    scratch_types=[
        pltpu.VMEM_SHARED((_OCH,), jnp.int32),
        pltpu.VMEM_SHARED((_OCH,), jnp.int32),
        pltpu.VMEM((_RT * _W,), jnp.int32),
        pltpu.VMEM((_XCH,), jnp.int32),
        pltpu.VMEM((_XCH,), jnp.int32),
        pltpu.VMEM((_XCH,), jnp.int32),
        pltpu.VMEM((_XCH,), jnp.int32),
        pltpu.VMEM((_XCH,), jnp.int32),
        pltpu.VMEM((_XCH,), jnp.int32),
        pltpu.VMEM((_XCH,), jnp.int32),
        pltpu.SemaphoreType.DMA,
        pltpu.SemaphoreType.DMA,
    ],
)
def _onehot_sc(x_hbm, common_hbm, out_hbm, img0, img1, zb, xb0, xb1, ib0,
               ib1, cb, ones_v, zeros_v, sem0, sem1):
    _body(x_hbm, common_hbm, out_hbm, img0, img1, zb, xb0, xb1, ib0, ib1,
          cb, ones_v, zeros_v, sem0, sem1)


def kernel(x):
    t = jnp.arange(_XCH, dtype=jnp.int32)
    common = (t // _F) * _W + (t % _F) * _C
    return _onehot_sc(x.reshape(-1), common).reshape(_B, _W)


# speedup vs baseline: 1.0519x; 1.0316x over previous
import functools

import jax
import jax.numpy as jnp
from jax import lax
from jax.experimental import pallas as pl
from jax.experimental.pallas import tpu as pltpu
from jax.experimental.pallas import tpu_sc as plsc

_B = 16384
_F = 26
_C = 100
_W = _F * _C
_NC = 2
_NS = 16
_L = 16
_ROWS_PER_C = _B // _NC

_SRB = 8
_SSL = 3
_SROWS = 384
_SNB = _SROWS // _SRB
_SXB = _SRB * _F
_SGRP = _SXB // _L

_PRB = 16
_PROWS = 128
_PNB = _PROWS // _PRB
_PXB = _PRB * _F
_PGRP = _PXB // _L
_PW = _PRB * _W
_PSLOT = _PW
_IMG = _NS * _PSLOT

_NIT = 3

_SROW0 = _NS * _SROWS


def _body(x_hbm, rb_hbm, cb_hbm, common_hbm, z_hbm, out_hbm, sb0, sb1, sb2, sb3,
          img0, sxbuf, pxbuf, rb, cbs, cb, ones_v, zeros_v, pi0,
          ss0, ss1, ss2, ss3, ps0):
    c = lax.axis_index("c")
    s = lax.axis_index("s")
    crow = c * _ROWS_PER_C
    sbase = s * _PSLOT
    srow0 = crow + s * _SROWS
    prow0 = crow + _SROW0 + s * _PROWS

    pltpu.sync_copy(x_hbm.at[pl.ds(srow0 * _F, _SROWS * _F)], sxbuf)
    pltpu.sync_copy(rb_hbm, rb)
    pltpu.sync_copy(cb_hbm, cbs)
    pltpu.sync_copy(common_hbm, cb)

    zeros = jnp.zeros((_L,), jnp.int32)
    ones = jnp.full((_L,), 1, jnp.int32)
    for g in range(_PGRP):
        ones_v[pl.ds(g * _L, _L)] = ones
        zeros_v[pl.ds(g * _L, _L)] = zeros

    sbufs = (sb0, sb1, sb2)
    ssems = (ss0, ss1, ss2)
    imgs = (img0,)
    psems = (ps0,)
    pidx = (pi0,)

    def zfill(i, carry):
        for b in sbufs:
            b[pl.ds(i * _L, _L)] = zeros
        return carry

    lax.fori_loop(0, _SRB * _W // _L, zfill, 0)
    for img in imgs:
        pltpu.sync_copy(z_hbm, img.at[pl.ds(sbase, _PSLOT)])

    def s_scatter(slot, it, val):
        xoff = it * _SXB
        for g in range(_SGRP):
            rv = rb[pl.ds(g * _L, _L)]
            cv = cbs[pl.ds(g * _L, _L)]
            xv = sxbuf[pl.ds(xoff + g * _L, _L)]
            plsc.store_scatter(sbufs[slot], [rv + cv + xv], val)

    def s_start(slot, it):
        pltpu.make_async_copy(
            sbufs[slot],
            out_hbm.at[pl.ds((srow0 + it * _SRB) * _W, _SRB * _W)],
            ssems[slot],
        ).start()

    def s_wait(slot):
        pltpu.make_async_copy(
            sbufs[slot], out_hbm.at[pl.ds(srow0 * _W, _SRB * _W)],
            ssems[slot],
        ).wait()

    def s_batch(slot, it, first):
        if not first:
            s_wait(slot)
            s_scatter(slot, it - _SSL, zeros)
        s_scatter(slot, it, ones)
        s_start(slot, it)

    def p_build_idx(slot, it):
        pltpu.sync_copy(x_hbm.at[pl.ds((prow0 + it * _PRB) * _F, _PXB)],
                        pxbuf)
        for g in range(_PGRP):
            cv = cb[pl.ds(g * _L, _L)]
            xv = pxbuf[pl.ds(g * _L, _L)]
            pidx[slot][pl.ds(g * _L, _L)] = cv + xv + sbase

    def p_start(slot, it):
        pltpu.make_async_copy(
            imgs[slot].at[pl.ds(sbase, _PW)],
            out_hbm.at[pl.ds((prow0 + it * _PRB) * _W, _PW)],
            psems[slot],
        ).start()

    def p_wait(slot):
        pltpu.make_async_copy(
            imgs[slot].at[pl.ds(sbase, _PW)],
            out_hbm.at[pl.ds(prow0 * _W, _PW)],
            psems[slot],
        ).wait()

    def p_batch(slot, it, first):
        if not first:
            p_wait(slot)
            pltpu.sync_copy(zeros_v, imgs[slot].at[pidx[slot]])
        p_build_idx(slot, it)
        pltpu.sync_copy(ones_v, imgs[slot].at[pidx[slot]])
        p_start(slot, it)

    for slot in range(_SSL):
        s_batch(slot, slot, True)
    p_batch(0, 0, True)

    def body(i, carry):
        sit0 = _SSL + i * 15
        pit0 = 1 + i * 2
        for t in range(15):
            s_batch(t % _SSL, sit0 + t, False)
            if t == 4:
                p_batch(0, pit0, False)
            elif t == 9:
                p_batch(0, pit0 + 1, False)
        return carry

    lax.fori_loop(0, _NIT, body, 0)
    p_batch(0, _PNB - 1, False)

    for slot in range(_SSL):
        s_wait(slot)
    p_wait(0)


@functools.partial(
    pl.kernel,
    out_type=jax.ShapeDtypeStruct((_B * _W,), jnp.int32),
    mesh=plsc.VectorSubcoreMesh(core_axis_name="c", subcore_axis_name="s"),
    compiler_params=pltpu.CompilerParams(needs_layout_passes=False),
    scratch_types=[
        pltpu.VMEM((_SRB * _W,), jnp.int32),
        pltpu.VMEM((_SRB * _W,), jnp.int32),
        pltpu.VMEM((_SRB * _W,), jnp.int32),
        pltpu.VMEM_SHARED((_IMG,), jnp.int32),
        pltpu.VMEM((_SROWS * _F,), jnp.int32),
        pltpu.VMEM((_PXB,), jnp.int32),
        pltpu.VMEM((_SXB,), jnp.int32),
        pltpu.VMEM((_SXB,), jnp.int32),
        pltpu.VMEM((_PXB,), jnp.int32),
        pltpu.VMEM((_PXB,), jnp.int32),
        pltpu.VMEM((_PXB,), jnp.int32),
        pltpu.VMEM((_PXB,), jnp.int32),
        pltpu.SemaphoreType.DMA,
        pltpu.SemaphoreType.DMA,
        pltpu.SemaphoreType.DMA,
        pltpu.SemaphoreType.DMA,
    ],
)
def _onehot_sc(x_hbm, rb_hbm, cb_hbm, common_hbm, z_hbm, out_hbm, sb0, sb1,
               sb2, img0, sxbuf, pxbuf, rb, cbs, cb, ones_v,
               zeros_v, pi0, ss0, ss1, ss2, ps0):
    _body(x_hbm, rb_hbm, cb_hbm, common_hbm, z_hbm, out_hbm, sb0, sb1, sb2,
          img0, sxbuf, pxbuf, rb, cbs, cb, ones_v, zeros_v, pi0,
          ss0, ss1, ss2, ps0)


def kernel(x):
    ts = jnp.arange(_SXB, dtype=jnp.int32)
    rbase = (ts // _F) * _W
    cbase = (ts % _F) * _C
    tp = jnp.arange(_PXB, dtype=jnp.int32)
    common = (tp // _F) * _W + (tp % _F) * _C
    zblk = jnp.zeros((_PSLOT,), jnp.int32)
    return _onehot_sc(x.reshape(-1), rbase, cbase, common,
                      zblk).reshape(_B, _W)


# speedup vs baseline: 1.7796x; 1.6919x over previous
import functools

import jax
import jax.numpy as jnp
from jax import lax
from jax.experimental import pallas as pl
from jax.experimental.pallas import tpu as pltpu
from jax.experimental.pallas import tpu_sc as plsc

_B = 16384
_F = 26
_C = 100
_W = _F * _C
_NC = 2
_NS = 16
_NW = _NC * _NS
_L = 16
_ROWS_PER_W = _B // _NW
_RB = 16
_NIT = _ROWS_PER_W // _RB
_XCH = _RB * _F
_XPW = _ROWS_PER_W * _F
_NGRP = _XCH // _L

_ZOFFS = tuple(range(0, _W - _L + 1, _L)) + (_W - _L,)


def _body(x_hbm, rbase_hbm, cbase_hbm, out_hbm, buf0, buf1, xbuf, rbuf, cbuf,
          sem0, sem1):
    wid = lax.axis_index("s") * _NC + lax.axis_index("c")
    row0 = wid * _ROWS_PER_W

    pltpu.sync_copy(x_hbm.at[pl.ds(wid * _XPW, _XPW)], xbuf)
    pltpu.sync_copy(rbase_hbm, rbuf)
    pltpu.sync_copy(cbase_hbm, cbuf)

    zeros = jnp.zeros((_L,), jnp.int32)
    ones = jnp.full((_L,), 1, jnp.int32)

    def zbody(r, carry):
        for c in _ZOFFS:
            buf0[r, pl.ds(c, _L)] = zeros
            buf1[r, pl.ds(c, _L)] = zeros
        return carry

    lax.fori_loop(0, _RB, zbody, 0)

    bufs = (buf0, buf1)
    sems = (sem0, sem1)

    def scatter(buf, it, val):
        xoff = it * _XCH
        for g in range(_NGRP):
            rv = rbuf[pl.ds(g * _L, _L)]
            cv = cbuf[pl.ds(g * _L, _L)]
            xv = xbuf[pl.ds(xoff + g * _L, _L)]
            plsc.store_scatter(buf, [rv, cv + xv], val)

    def start_out(b, it):
        pltpu.make_async_copy(
            bufs[b], out_hbm.at[pl.ds(row0 + it * _RB, _RB)], sems[b]
        ).start()

    def wait_out(b):
        pltpu.make_async_copy(
            bufs[b], out_hbm.at[pl.ds(row0, _RB)], sems[b]
        ).wait()

    for b in range(2):
        scatter(bufs[b], b, ones)
        start_out(b, b)

    def body(i, carry):
        it0 = 2 + i * 2
        for b in range(2):
            it = it0 + b
            wait_out(b)
            scatter(bufs[b], it - 2, zeros)
            scatter(bufs[b], it, ones)
            start_out(b, it)
        return carry

    lax.fori_loop(0, (_NIT - 2) // 2, body, 0)

    for b in range(2):
        wait_out(b)


@functools.partial(
    pl.kernel,
    out_type=jax.ShapeDtypeStruct((_B, _W), jnp.int32),
    mesh=plsc.VectorSubcoreMesh(core_axis_name="c", subcore_axis_name="s"),
    compiler_params=pltpu.CompilerParams(needs_layout_passes=False),
    scratch_types=[
        pltpu.VMEM((_RB, _W), jnp.int32),
        pltpu.VMEM((_RB, _W), jnp.int32),
        pltpu.VMEM((_XPW,), jnp.int32),
        pltpu.VMEM((_XCH,), jnp.int32),
        pltpu.VMEM((_XCH,), jnp.int32),
        pltpu.SemaphoreType.DMA,
        pltpu.SemaphoreType.DMA,
    ],
)
def _onehot_sc(x_hbm, rbase_hbm, cbase_hbm, out_hbm, buf0, buf1, xbuf, rbuf,
               cbuf, sem0, sem1):
    _body(x_hbm, rbase_hbm, cbase_hbm, out_hbm, buf0, buf1, xbuf, rbuf, cbuf,
          sem0, sem1)


def kernel(x):
    t = jnp.arange(_XCH, dtype=jnp.int32)
    rbase = t // _F
    cbase = (t % _F) * _C
    return _onehot_sc(x.reshape(-1), rbase, cbase)


# speedup vs baseline: 1.7885x; 1.0050x over previous
import functools

import jax
import jax.numpy as jnp
from jax import lax
from jax.experimental import pallas as pl
from jax.experimental.pallas import tpu as pltpu
from jax.experimental.pallas import tpu_sc as plsc

_B = 16384
_F = 26
_C = 100
_W = _F * _C
_NC = 2
_NS = 16
_NW = _NC * _NS
_L = 16
_ROWS_PER_W = _B // _NW
_RB = 8
_NSL = 4
_NIT = _ROWS_PER_W // _RB
_XCH = _RB * _F
_XPW = _ROWS_PER_W * _F
_NGRP = _XCH // _L

_ZOFFS = tuple(range(0, _W - _L + 1, _L)) + (_W - _L,)


def _body(x_hbm, rbase_hbm, cbase_hbm, out_hbm, buf0, buf1, buf2, buf3,
          xbuf, rbuf, cbuf, sem0, sem1, sem2, sem3):
    wid = lax.axis_index("s") * _NC + lax.axis_index("c")
    row0 = wid * _ROWS_PER_W

    pltpu.sync_copy(x_hbm.at[pl.ds(wid * _XPW, _XPW)], xbuf)
    pltpu.sync_copy(rbase_hbm, rbuf)
    pltpu.sync_copy(cbase_hbm, cbuf)

    zeros = jnp.zeros((_L,), jnp.int32)
    ones = jnp.full((_L,), 1, jnp.int32)

    bufs = (buf0, buf1, buf2, buf3)
    sems = (sem0, sem1, sem2, sem3)

    def zbody(r, carry):
        for c in _ZOFFS:
            for b in bufs:
                b[r, pl.ds(c, _L)] = zeros
        return carry

    lax.fori_loop(0, _RB, zbody, 0)

    def scatter(buf, it, val):
        xoff = it * _XCH
        for g in range(_NGRP):
            rv = rbuf[pl.ds(g * _L, _L)]
            cv = cbuf[pl.ds(g * _L, _L)]
            xv = xbuf[pl.ds(xoff + g * _L, _L)]
            plsc.store_scatter(buf, [rv, cv + xv], val)

    def start_out(b, it):
        pltpu.make_async_copy(
            bufs[b], out_hbm.at[pl.ds(row0 + it * _RB, _RB)], sems[b]
        ).start()

    def wait_out(b):
        pltpu.make_async_copy(
            bufs[b], out_hbm.at[pl.ds(row0, _RB)], sems[b]
        ).wait()

    for b in range(_NSL):
        scatter(bufs[b], b, ones)
        start_out(b, b)

    def body(i, carry):
        it0 = _NSL + i * _NSL
        for b in range(_NSL):
            it = it0 + b
            wait_out(b)
            scatter(bufs[b], it - _NSL, zeros)
            scatter(bufs[b], it, ones)
            start_out(b, it)
        return carry

    lax.fori_loop(0, (_NIT - _NSL) // _NSL, body, 0)

    for b in range(_NSL):
        wait_out(b)


@functools.partial(
    pl.kernel,
    out_type=jax.ShapeDtypeStruct((_B, _W), jnp.int32),
    mesh=plsc.VectorSubcoreMesh(core_axis_name="c", subcore_axis_name="s"),
    compiler_params=pltpu.CompilerParams(needs_layout_passes=False),
    scratch_types=[
        pltpu.VMEM((_RB, _W), jnp.int32),
        pltpu.VMEM((_RB, _W), jnp.int32),
        pltpu.VMEM((_RB, _W), jnp.int32),
        pltpu.VMEM((_RB, _W), jnp.int32),
        pltpu.VMEM((_XPW,), jnp.int32),
        pltpu.VMEM((_XCH,), jnp.int32),
        pltpu.VMEM((_XCH,), jnp.int32),
        pltpu.SemaphoreType.DMA,
        pltpu.SemaphoreType.DMA,
        pltpu.SemaphoreType.DMA,
        pltpu.SemaphoreType.DMA,
    ],
)
def _onehot_sc(x_hbm, rbase_hbm, cbase_hbm, out_hbm, buf0, buf1, buf2, buf3,
               xbuf, rbuf, cbuf, sem0, sem1, sem2, sem3):
    _body(x_hbm, rbase_hbm, cbase_hbm, out_hbm, buf0, buf1, buf2, buf3,
          xbuf, rbuf, cbuf, sem0, sem1, sem2, sem3)


def kernel(x):
    t = jnp.arange(_XCH, dtype=jnp.int32)
    rbase = t // _F
    cbase = (t % _F) * _C
    return _onehot_sc(x.reshape(-1), rbase, cbase)
